# baseline (speedup 1.0000x reference)
import functools

import jax
import jax.numpy as jnp
from jax import lax
from jax.experimental import pallas as pl
from jax.experimental.pallas import tpu as pltpu
from jax.experimental.pallas import tpu_sc as plsc

_N = 10000
_E = 320000
_DIN = 128
_DH = 256
_QW = 64
_NC = 2
_NS = 16
_CH = 128
_NP = 10240
_EP = 327680
_RPT = _NP // _NS
_MSG_CHUNKS = _EP // (_NS * _CH)
_DEG_CHUNKS = _EP // (_NC * _NS * _CH)
_DEGW = 16


@functools.cache
def _sc_mesh():
    return plsc.VectorSubcoreMesh(core_axis_name="c", subcore_axis_name="s",
                                  num_cores=_NC, num_subcores=_NS)


def _deg_body(dst_hbm, ones_hbm, zeros_hbm, out0_hbm, out1_hbm,
              dstb_v, ones_v, acc_sh, sem):
    cid = lax.axis_index("c")
    sid = lax.axis_index("s")
    wid = cid * _NS + sid
    pltpu.sync_copy(dst_hbm.at[wid], dstb_v)
    pltpu.sync_copy(ones_hbm, ones_v)
    stripe = pl.ds(sid * _RPT, _RPT)
    pltpu.sync_copy(zeros_hbm.at[stripe], acc_sh.at[stripe])
    plsc.subcore_barrier()

    grp = 8

    def group(g, carry):
        ds = []
        for b in range(grp):
            c = g * grp + b
            ds.append(pltpu.async_copy(
                ones_v, acc_sh.at[dstb_v.at[c]], sem, add=True))
        for d in ds:
            d.wait()
        return carry

    lax.fori_loop(0, _DEG_CHUNKS // grp, group, 0)
    plsc.subcore_barrier()

    @pl.when(cid == 0)
    def _():
        pltpu.sync_copy(acc_sh.at[stripe], out0_hbm.at[stripe])

    @pl.when(cid == 1)
    def _():
        pltpu.sync_copy(acc_sh.at[stripe], out1_hbm.at[stripe])


@functools.cache
def _deg_call():
    return pl.kernel(
        _deg_body,
        out_type=(jax.ShapeDtypeStruct((_NP, _DEGW), jnp.float32),
                  jax.ShapeDtypeStruct((_NP, _DEGW), jnp.float32)),
        mesh=_sc_mesh(),
        compiler_params=pltpu.CompilerParams(use_tc_tiling_on_sc=False),
        scratch_types=[
            pltpu.VMEM((_DEG_CHUNKS, _CH), jnp.int32),
            pltpu.VMEM((_CH, _DEGW), jnp.float32),
            pltpu.VMEM_SHARED((_NP, _DEGW), jnp.float32),
            pltpu.SemaphoreType.DMA,
        ],
    )


_NBUF = 5
_LEAD = 3
_LAG = _NBUF - _LEAD


def _msg_body(src_hbm, dst_hbm, hA_hbm, hB_hbm, outA_hbm, outB_hbm,
              srcb_v, dstb_v, stage_v, acc_sh, gsem, ssem):
    cid = lax.axis_index("c")
    sid = lax.axis_index("s")
    stripe = pl.ds(sid * _RPT, _RPT)
    pltpu.sync_copy(src_hbm.at[sid], srcb_v)
    pltpu.sync_copy(dst_hbm.at[sid], dstb_v)

    def pipeline(h_hbm):
        for b in range(_LEAD):
            pltpu.async_copy(h_hbm.at[srcb_v.at[b]], stage_v.at[b],
                             gsem.at[b])

        def group(k, carry):
            for b in range(_NBUF):
                c = k * _NBUF + b
                bn = (b + _LEAD) % _NBUF
                pltpu.make_async_copy(h_hbm.at[srcb_v.at[c]], stage_v.at[b],
                                      gsem.at[b]).wait()
                pltpu.async_copy(stage_v.at[b], acc_sh.at[dstb_v.at[c]],
                                 ssem.at[b], add=True)

                def _wait_prev():
                    pltpu.make_async_copy(
                        stage_v.at[bn], acc_sh.at[dstb_v.at[c - _LAG]],
                        ssem.at[bn]).wait()

                if b < _LAG:
                    @pl.when(k > 0)
                    def _():
                        _wait_prev()
                else:
                    _wait_prev()

                @pl.when(c + _LEAD < _MSG_CHUNKS)
                def _():
                    pltpu.async_copy(h_hbm.at[srcb_v.at[c + _LEAD]],
                                     stage_v.at[bn], gsem.at[bn])
            return carry

        lax.fori_loop(0, _MSG_CHUNKS // _NBUF, group, 0)
        for b in range(_LEAD, _NBUF):
            c = _MSG_CHUNKS - _NBUF + b
            pltpu.make_async_copy(stage_v.at[b], acc_sh.at[dstb_v.at[c]],
                                  ssem.at[b]).wait()

    @pl.when(cid == 0)
    def _():
        pltpu.sync_copy(hA_hbm.at[stripe], acc_sh.at[stripe])

    @pl.when(cid == 1)
    def _():
        pltpu.sync_copy(hB_hbm.at[stripe], acc_sh.at[stripe])

    plsc.subcore_barrier()

    @pl.when(cid == 0)
    def _():
        pipeline(hA_hbm)

    @pl.when(cid == 1)
    def _():
        pipeline(hB_hbm)

    plsc.subcore_barrier()

    @pl.when(cid == 0)
    def _():
        pltpu.sync_copy(acc_sh.at[stripe], outA_hbm.at[stripe])

    @pl.when(cid == 1)
    def _():
        pltpu.sync_copy(acc_sh.at[stripe], outB_hbm.at[stripe])


@functools.cache
def _msg_call():
    return pl.kernel(
        _msg_body,
        out_type=(jax.ShapeDtypeStruct((_NP, _QW), jnp.float32),
                  jax.ShapeDtypeStruct((_NP, _QW), jnp.float32)),
        mesh=_sc_mesh(),
        compiler_params=pltpu.CompilerParams(use_tc_tiling_on_sc=False),
        scratch_types=[
            pltpu.VMEM((_MSG_CHUNKS, _CH), jnp.int32),
            pltpu.VMEM((_MSG_CHUNKS, _CH), jnp.int32),
            pltpu.VMEM((_NBUF, _CH, _QW), jnp.float32),
            pltpu.VMEM_SHARED((_NP, _QW), jnp.float32),
            pltpu.SemaphoreType.DMA((_NBUF,)),
            pltpu.SemaphoreType.DMA((_NBUF,)),
        ],
    )


def _msg_layer(src_m, dst_m, hq):
    s0, s1 = _msg_call()(src_m, dst_m, hq[0], hq[1])
    s2, s3 = _msg_call()(src_m, dst_m, hq[2], hq[3])
    return (s0, s1, s2, s3)


def _dinv(i, d0, d1, blk):
    deg = d0[:, :1] + d1[:, :1] + 1.0
    rows = i * blk + lax.broadcasted_iota(jnp.int32, (blk, 1), 0)
    return jnp.where(rows < _N, lax.rsqrt(deg), 0.0)


def _mm1_body(x_ref, w_ref, d0_ref, d1_ref, *out_refs):
    i = pl.program_id(0)
    h = jnp.dot(x_ref[...], w_ref[...], preferred_element_type=jnp.float32)
    ht = h * _dinv(i, d0_ref[...], d1_ref[...], _RPT)
    for q, o_ref in enumerate(out_refs):
        o_ref[...] = ht[:, q * _QW:(q + 1) * _QW]


_mm1_call = pl.pallas_call(
    _mm1_body,
    grid=(_NS,),
    in_specs=[
        pl.BlockSpec((_RPT, _DIN), lambda i: (i, 0)),
        pl.BlockSpec((_DIN, _DH), lambda i: (0, 0)),
        pl.BlockSpec((_RPT, _DEGW), lambda i: (i, 0)),
        pl.BlockSpec((_RPT, _DEGW), lambda i: (i, 0)),
    ],
    out_specs=tuple(pl.BlockSpec((_RPT, _QW), lambda i: (i, 0))
                    for _ in range(4)),
    out_shape=tuple(jax.ShapeDtypeStruct((_NP, _QW), jnp.float32)
                    for _ in range(4)),
)


def _mm2_body(s0, s1, s2, s3, d0_ref, d1_ref, b_ref, w_ref, *out_refs):
    i = pl.program_id(0)
    dv = _dinv(i, d0_ref[...], d1_ref[...], _RPT)
    s = jnp.concatenate([s0[...], s1[...], s2[...], s3[...]], axis=1)
    z = jnp.maximum(s * dv + b_ref[...], 0.0)
    h2 = jnp.dot(z, w_ref[...], preferred_element_type=jnp.float32)
    ht2 = h2 * dv
    for q, o_ref in enumerate(out_refs):
        o_ref[...] = ht2[:, q * _QW:(q + 1) * _QW]


_mm2_call = pl.pallas_call(
    _mm2_body,
    grid=(_NS,),
    in_specs=[pl.BlockSpec((_RPT, _QW), lambda i: (i, 0)) for _ in range(4)]
    + [
        pl.BlockSpec((_RPT, _DEGW), lambda i: (i, 0)),
        pl.BlockSpec((_RPT, _DEGW), lambda i: (i, 0)),
        pl.BlockSpec((1, _DH), lambda i: (0, 0)),
        pl.BlockSpec((_DH, _DH), lambda i: (0, 0)),
    ],
    out_specs=tuple(pl.BlockSpec((_RPT, _QW), lambda i: (i, 0))
                    for _ in range(4)),
    out_shape=tuple(jax.ShapeDtypeStruct((_NP, _QW), jnp.float32)
                    for _ in range(4)),
)

_FBLK = 1000


def _fin_body(s0, s1, s2, s3, d0_ref, d1_ref, b_ref, out_ref):
    i = pl.program_id(0)
    dv = _dinv(i, d0_ref[...], d1_ref[...], _FBLK)
    s = jnp.concatenate([s0[...], s1[...], s2[...], s3[...]], axis=1)
    out_ref[...] = jnp.maximum(s * dv + b_ref[...], 0.0)


_fin_call = pl.pallas_call(
    _fin_body,
    grid=(_N // _FBLK,),
    in_specs=[pl.BlockSpec((_FBLK, _QW), lambda i: (i, 0)) for _ in range(4)]
    + [
        pl.BlockSpec((_FBLK, _DEGW), lambda i: (i, 0)),
        pl.BlockSpec((_FBLK, _DEGW), lambda i: (i, 0)),
        pl.BlockSpec((1, _DH), lambda i: (0, 0)),
    ],
    out_specs=pl.BlockSpec((_FBLK, _DH), lambda i: (i, 0)),
    out_shape=jax.ShapeDtypeStruct((_N, _DH), jnp.float32),
)


def kernel(x, edge_index, W1, b1, W2, b2):
    f32 = jnp.float32
    npad = _EP - _E
    padn = _N + (jnp.arange(npad, dtype=jnp.int32) % 224)
    src_p = jnp.concatenate([edge_index[0], padn])
    dst_p = jnp.concatenate([edge_index[1], padn])
    src_m = src_p.reshape(_NS, _MSG_CHUNKS, _CH)
    dst_m = dst_p.reshape(_NS, _MSG_CHUNKS, _CH)
    dst_d = dst_p.reshape(_NC * _NS, _DEG_CHUNKS, _CH)
    x_p = jnp.zeros((_NP, _DIN), f32).at[:_N].set(x)
    ones = jnp.ones((_CH, _DEGW), f32)
    zeros = jnp.zeros((_NP, _DEGW), f32)

    deg0, deg1 = _deg_call()(dst_d, ones, zeros)
    ht1 = _mm1_call(x_p, W1, deg0, deg1)
    s1 = _msg_layer(src_m, dst_m, ht1)
    ht2 = _mm2_call(*s1, deg0, deg1, b1.reshape(1, _DH), W2)
    s2 = _msg_layer(src_m, dst_m, ht2)
    return _fin_call(*s2, deg0, deg1, b2.reshape(1, _DH))


# speedup vs baseline: 1.0244x; 1.0244x over previous
import functools

import jax
import jax.numpy as jnp
from jax import lax
from jax.experimental import pallas as pl
from jax.experimental.pallas import tpu as pltpu
from jax.experimental.pallas import tpu_sc as plsc

_N = 10000
_E = 320000
_DIN = 128
_DH = 256
_QW = 64
_NC = 2
_NS = 16
_CH = 128
_NP = 10240
_EP = 327680
_RPT = _NP // _NS
_MSG_CHUNKS = _EP // (_NS * _CH)
_DEG_CHUNKS = _EP // (_NC * _NS * _CH)
_DEGW = 16


@functools.cache
def _sc_mesh():
    return plsc.VectorSubcoreMesh(core_axis_name="c", subcore_axis_name="s",
                                  num_cores=_NC, num_subcores=_NS)


def _deg_body(dst_hbm, ones_hbm, zeros_hbm, out0_hbm, out1_hbm,
              dstb_v, ones_v, acc_sh, sem):
    cid = lax.axis_index("c")
    sid = lax.axis_index("s")
    wid = cid * _NS + sid
    pltpu.sync_copy(dst_hbm.at[wid], dstb_v)
    pltpu.sync_copy(ones_hbm, ones_v)
    stripe = pl.ds(sid * _RPT, _RPT)
    pltpu.sync_copy(zeros_hbm.at[stripe], acc_sh.at[stripe])
    plsc.subcore_barrier()

    grp = 8

    def group(g, carry):
        ds = []
        for b in range(grp):
            c = g * grp + b
            ds.append(pltpu.async_copy(
                ones_v, acc_sh.at[dstb_v.at[c]], sem, add=True))
        for d in ds:
            d.wait()
        return carry

    lax.fori_loop(0, _DEG_CHUNKS // grp, group, 0)
    plsc.subcore_barrier()

    @pl.when(cid == 0)
    def _():
        pltpu.sync_copy(acc_sh.at[stripe], out0_hbm.at[stripe])

    @pl.when(cid == 1)
    def _():
        pltpu.sync_copy(acc_sh.at[stripe], out1_hbm.at[stripe])


@functools.cache
def _deg_call():
    return pl.kernel(
        _deg_body,
        out_type=(jax.ShapeDtypeStruct((_NP, _DEGW), jnp.float32),
                  jax.ShapeDtypeStruct((_NP, _DEGW), jnp.float32)),
        mesh=_sc_mesh(),
        compiler_params=pltpu.CompilerParams(use_tc_tiling_on_sc=False),
        scratch_types=[
            pltpu.VMEM((_DEG_CHUNKS, _CH), jnp.int32),
            pltpu.VMEM((_CH, _DEGW), jnp.float32),
            pltpu.VMEM_SHARED((_NP, _DEGW), jnp.float32),
            pltpu.SemaphoreType.DMA,
        ],
    )


_NBUF = 4


def _msg_body(src_hbm, dst_hbm, h0_hbm, h1_hbm, h2_hbm, h3_hbm,
              out0_hbm, out1_hbm, out2_hbm, out3_hbm,
              srcb_v, dstb_v, stage_v, acc_sh, gsem, ssem):
    cid = lax.axis_index("c")
    sid = lax.axis_index("s")
    stripe = pl.ds(sid * _RPT, _RPT)
    pltpu.sync_copy(src_hbm.at[sid], srcb_v)
    pltpu.sync_copy(dst_hbm.at[sid], dstb_v)

    def pipeline(h_hbm):
        for b in range(_NBUF):
            pltpu.async_copy(h_hbm.at[srcb_v.at[b]], stage_v.at[b],
                             gsem.at[b])

        def group(k, carry):
            for b in range(_NBUF):
                c = k * _NBUF + b
                pltpu.make_async_copy(h_hbm.at[srcb_v.at[c]], stage_v.at[b],
                                      gsem.at[b]).wait()
                pltpu.async_copy(stage_v.at[b], acc_sh.at[dstb_v.at[c]],
                                 ssem.at[b], add=True).wait()

                @pl.when(c + _NBUF < _MSG_CHUNKS)
                def _():
                    pltpu.async_copy(h_hbm.at[srcb_v.at[c + _NBUF]],
                                     stage_v.at[b], gsem.at[b])
            return carry

        lax.fori_loop(0, _MSG_CHUNKS // _NBUF, group, 0)

    for hA, hB, outA, outB in ((h0_hbm, h1_hbm, out0_hbm, out1_hbm),
                               (h2_hbm, h3_hbm, out2_hbm, out3_hbm)):
        @pl.when(cid == 0)
        def _():
            pltpu.sync_copy(hA.at[stripe], acc_sh.at[stripe])

        @pl.when(cid == 1)
        def _():
            pltpu.sync_copy(hB.at[stripe], acc_sh.at[stripe])

        plsc.subcore_barrier()

        @pl.when(cid == 0)
        def _():
            pipeline(hA)

        @pl.when(cid == 1)
        def _():
            pipeline(hB)

        plsc.subcore_barrier()

        @pl.when(cid == 0)
        def _():
            pltpu.sync_copy(acc_sh.at[stripe], outA.at[stripe])

        @pl.when(cid == 1)
        def _():
            pltpu.sync_copy(acc_sh.at[stripe], outB.at[stripe])

        plsc.subcore_barrier()


@functools.cache
def _msg_call():
    return pl.kernel(
        _msg_body,
        out_type=tuple(jax.ShapeDtypeStruct((_NP, _QW), jnp.float32)
                       for _ in range(4)),
        mesh=_sc_mesh(),
        compiler_params=pltpu.CompilerParams(use_tc_tiling_on_sc=False),
        scratch_types=[
            pltpu.VMEM((_MSG_CHUNKS, _CH), jnp.int32),
            pltpu.VMEM((_MSG_CHUNKS, _CH), jnp.int32),
            pltpu.VMEM((_NBUF, _CH, _QW), jnp.float32),
            pltpu.VMEM_SHARED((_NP, _QW), jnp.float32),
            pltpu.SemaphoreType.DMA((_NBUF,)),
            pltpu.SemaphoreType.DMA((_NBUF,)),
        ],
    )


def _msg_layer(src_m, dst_m, hq):
    return _msg_call()(src_m, dst_m, *hq)


def _dinv(i, d0, d1, blk):
    deg = d0[:, :1] + d1[:, :1] + 1.0
    rows = i * blk + lax.broadcasted_iota(jnp.int32, (blk, 1), 0)
    return jnp.where(rows < _N, lax.rsqrt(deg), 0.0)


def _mm1_body(x_ref, w_ref, d0_ref, d1_ref, *out_refs):
    i = pl.program_id(0)
    h = jnp.dot(x_ref[...], w_ref[...], preferred_element_type=jnp.float32)
    ht = h * _dinv(i, d0_ref[...], d1_ref[...], _RPT)
    for q, o_ref in enumerate(out_refs):
        o_ref[...] = ht[:, q * _QW:(q + 1) * _QW]


_mm1_call = pl.pallas_call(
    _mm1_body,
    grid=(_NS,),
    in_specs=[
        pl.BlockSpec((_RPT, _DIN), lambda i: (i, 0)),
        pl.BlockSpec((_DIN, _DH), lambda i: (0, 0)),
        pl.BlockSpec((_RPT, _DEGW), lambda i: (i, 0)),
        pl.BlockSpec((_RPT, _DEGW), lambda i: (i, 0)),
    ],
    out_specs=tuple(pl.BlockSpec((_RPT, _QW), lambda i: (i, 0))
                    for _ in range(4)),
    out_shape=tuple(jax.ShapeDtypeStruct((_NP, _QW), jnp.float32)
                    for _ in range(4)),
)


def _mm2_body(s0, s1, s2, s3, d0_ref, d1_ref, b_ref, w_ref, *out_refs):
    i = pl.program_id(0)
    dv = _dinv(i, d0_ref[...], d1_ref[...], _RPT)
    s = jnp.concatenate([s0[...], s1[...], s2[...], s3[...]], axis=1)
    z = jnp.maximum(s * dv + b_ref[...], 0.0)
    h2 = jnp.dot(z, w_ref[...], preferred_element_type=jnp.float32)
    ht2 = h2 * dv
    for q, o_ref in enumerate(out_refs):
        o_ref[...] = ht2[:, q * _QW:(q + 1) * _QW]


_mm2_call = pl.pallas_call(
    _mm2_body,
    grid=(_NS,),
    in_specs=[pl.BlockSpec((_RPT, _QW), lambda i: (i, 0)) for _ in range(4)]
    + [
        pl.BlockSpec((_RPT, _DEGW), lambda i: (i, 0)),
        pl.BlockSpec((_RPT, _DEGW), lambda i: (i, 0)),
        pl.BlockSpec((1, _DH), lambda i: (0, 0)),
        pl.BlockSpec((_DH, _DH), lambda i: (0, 0)),
    ],
    out_specs=tuple(pl.BlockSpec((_RPT, _QW), lambda i: (i, 0))
                    for _ in range(4)),
    out_shape=tuple(jax.ShapeDtypeStruct((_NP, _QW), jnp.float32)
                    for _ in range(4)),
)

_FBLK = 1000


def _fin_body(s0, s1, s2, s3, d0_ref, d1_ref, b_ref, out_ref):
    i = pl.program_id(0)
    dv = _dinv(i, d0_ref[...], d1_ref[...], _FBLK)
    s = jnp.concatenate([s0[...], s1[...], s2[...], s3[...]], axis=1)
    out_ref[...] = jnp.maximum(s * dv + b_ref[...], 0.0)


_fin_call = pl.pallas_call(
    _fin_body,
    grid=(_N // _FBLK,),
    in_specs=[pl.BlockSpec((_FBLK, _QW), lambda i: (i, 0)) for _ in range(4)]
    + [
        pl.BlockSpec((_FBLK, _DEGW), lambda i: (i, 0)),
        pl.BlockSpec((_FBLK, _DEGW), lambda i: (i, 0)),
        pl.BlockSpec((1, _DH), lambda i: (0, 0)),
    ],
    out_specs=pl.BlockSpec((_FBLK, _DH), lambda i: (i, 0)),
    out_shape=jax.ShapeDtypeStruct((_N, _DH), jnp.float32),
)


def kernel(x, edge_index, W1, b1, W2, b2):
    f32 = jnp.float32
    npad = _EP - _E
    padn = _N + (jnp.arange(npad, dtype=jnp.int32) % 224)
    src_p = jnp.concatenate([edge_index[0], padn])
    dst_p = jnp.concatenate([edge_index[1], padn])
    src_m = src_p.reshape(_NS, _MSG_CHUNKS, _CH)
    dst_m = dst_p.reshape(_NS, _MSG_CHUNKS, _CH)
    dst_d = dst_p.reshape(_NC * _NS, _DEG_CHUNKS, _CH)
    x_p = jnp.zeros((_NP, _DIN), f32).at[:_N].set(x)
    ones = jnp.ones((_CH, _DEGW), f32)
    zeros = jnp.zeros((_NP, _DEGW), f32)

    deg0, deg1 = _deg_call()(dst_d, ones, zeros)
    ht1 = _mm1_call(x_p, W1, deg0, deg1)
    s1 = _msg_layer(src_m, dst_m, ht1)
    ht2 = _mm2_call(*s1, deg0, deg1, b1.reshape(1, _DH), W2)
    s2 = _msg_layer(src_m, dst_m, ht2)
    return _fin_call(*s2, deg0, deg1, b2.reshape(1, _DH))


# speedup vs baseline: 1.0331x; 1.0085x over previous
import functools

import jax
import jax.numpy as jnp
from jax import lax
from jax.experimental import pallas as pl
from jax.experimental.pallas import tpu as pltpu
from jax.experimental.pallas import tpu_sc as plsc

_N = 10000
_E = 320000
_DIN = 128
_DH = 256
_QW = 64
_NC = 2
_NS = 16
_CH = 128
_NP = 10240
_EP = 327680
_RPT = _NP // _NS
_MSG_CHUNKS = _EP // (_NS * _CH)
_DEG_CHUNKS = _EP // (_NC * _NS * _CH)
_DEGW = 16


@functools.cache
def _sc_mesh():
    return plsc.VectorSubcoreMesh(core_axis_name="c", subcore_axis_name="s",
                                  num_cores=_NC, num_subcores=_NS)


def _deg_body(dst_hbm, ones_hbm, zeros_hbm, out0_hbm, out1_hbm,
              dstb_v, ones_v, acc_sh, sem):
    cid = lax.axis_index("c")
    sid = lax.axis_index("s")
    wid = cid * _NS + sid
    pltpu.sync_copy(dst_hbm.at[wid], dstb_v)
    pltpu.sync_copy(ones_hbm, ones_v)
    stripe = pl.ds(sid * _RPT, _RPT)
    pltpu.sync_copy(zeros_hbm.at[stripe], acc_sh.at[stripe])
    plsc.subcore_barrier()

    grp = 8

    def group(g, carry):
        ds = []
        for b in range(grp):
            c = g * grp + b
            ds.append(pltpu.async_copy(
                ones_v, acc_sh.at[dstb_v.at[c]], sem, add=True))
        for d in ds:
            d.wait()
        return carry

    lax.fori_loop(0, _DEG_CHUNKS // grp, group, 0)
    plsc.subcore_barrier()

    @pl.when(cid == 0)
    def _():
        pltpu.sync_copy(acc_sh.at[stripe], out0_hbm.at[stripe])

    @pl.when(cid == 1)
    def _():
        pltpu.sync_copy(acc_sh.at[stripe], out1_hbm.at[stripe])


@functools.cache
def _deg_call():
    return pl.kernel(
        _deg_body,
        out_type=(jax.ShapeDtypeStruct((_NP, _DEGW), jnp.float32),
                  jax.ShapeDtypeStruct((_NP, _DEGW), jnp.float32)),
        mesh=_sc_mesh(),
        compiler_params=pltpu.CompilerParams(use_tc_tiling_on_sc=False),
        scratch_types=[
            pltpu.VMEM((_DEG_CHUNKS, _CH), jnp.int32),
            pltpu.VMEM((_CH, _DEGW), jnp.float32),
            pltpu.VMEM_SHARED((_NP, _DEGW), jnp.float32),
            pltpu.SemaphoreType.DMA,
        ],
    )


_NBUF = 4


def _msg_body(src_hbm, dst_hbm, h0_hbm, h1_hbm, h2_hbm, h3_hbm,
              out0_hbm, out1_hbm, out2_hbm, out3_hbm,
              srcb_v, dstb_v, stage_v, acc_sh, gsem, ssem):
    cid = lax.axis_index("c")
    sid = lax.axis_index("s")
    stripe = pl.ds(sid * _RPT, _RPT)
    pltpu.sync_copy(src_hbm.at[sid], srcb_v)
    pltpu.sync_copy(dst_hbm.at[sid], dstb_v)

    def pipeline(h_hbm):
        for b in range(_NBUF):
            pltpu.async_copy(h_hbm.at[srcb_v.at[b]], stage_v.at[b],
                             gsem.at[b])

        def group(k, carry):
            for b in range(_NBUF):
                c = k * _NBUF + b
                pltpu.make_async_copy(h_hbm.at[srcb_v.at[c]], stage_v.at[b],
                                      gsem.at[b]).wait()
                pltpu.async_copy(stage_v.at[b], acc_sh.at[dstb_v.at[c]],
                                 ssem.at[b], add=True).wait()

                @pl.when(c + _NBUF < _MSG_CHUNKS)
                def _():
                    pltpu.async_copy(h_hbm.at[srcb_v.at[c + _NBUF]],
                                     stage_v.at[b], gsem.at[b])
            return carry

        lax.fori_loop(0, _MSG_CHUNKS // _NBUF, group, 0)

    for p, (hA, hB, outA, outB) in enumerate(
            ((h0_hbm, h1_hbm, out0_hbm, out1_hbm),
             (h2_hbm, h3_hbm, out2_hbm, out3_hbm))):
        @pl.when(cid == 0)
        def _():
            if p:
                pltpu.sync_copy(acc_sh.at[stripe], out0_hbm.at[stripe])
            pltpu.sync_copy(hA.at[stripe], acc_sh.at[stripe])

        @pl.when(cid == 1)
        def _():
            if p:
                pltpu.sync_copy(acc_sh.at[stripe], out1_hbm.at[stripe])
            pltpu.sync_copy(hB.at[stripe], acc_sh.at[stripe])

        plsc.subcore_barrier()

        @pl.when(cid == 0)
        def _():
            pipeline(hA)

        @pl.when(cid == 1)
        def _():
            pipeline(hB)

        plsc.subcore_barrier()

    @pl.when(cid == 0)
    def _():
        pltpu.sync_copy(acc_sh.at[stripe], out2_hbm.at[stripe])

    @pl.when(cid == 1)
    def _():
        pltpu.sync_copy(acc_sh.at[stripe], out3_hbm.at[stripe])


@functools.cache
def _msg_call():
    return pl.kernel(
        _msg_body,
        out_type=tuple(jax.ShapeDtypeStruct((_NP, _QW), jnp.float32)
                       for _ in range(4)),
        mesh=_sc_mesh(),
        compiler_params=pltpu.CompilerParams(use_tc_tiling_on_sc=False),
        scratch_types=[
            pltpu.VMEM((_MSG_CHUNKS, _CH), jnp.int32),
            pltpu.VMEM((_MSG_CHUNKS, _CH), jnp.int32),
            pltpu.VMEM((_NBUF, _CH, _QW), jnp.float32),
            pltpu.VMEM_SHARED((_NP, _QW), jnp.float32),
            pltpu.SemaphoreType.DMA((_NBUF,)),
            pltpu.SemaphoreType.DMA((_NBUF,)),
        ],
    )


def _msg_layer(src_m, dst_m, hq):
    return _msg_call()(src_m, dst_m, *hq)


def _dinv(i, d0, d1, blk):
    deg = d0[:, :1] + d1[:, :1] + 1.0
    rows = i * blk + lax.broadcasted_iota(jnp.int32, (blk, 1), 0)
    return jnp.where(rows < _N, lax.rsqrt(deg), 0.0)


def _mm1_body(x_ref, w_ref, d0_ref, d1_ref, *out_refs):
    i = pl.program_id(0)
    h = jnp.dot(x_ref[...], w_ref[...], preferred_element_type=jnp.float32)
    ht = h * _dinv(i, d0_ref[...], d1_ref[...], _RPT)
    for q, o_ref in enumerate(out_refs):
        o_ref[...] = ht[:, q * _QW:(q + 1) * _QW]


_mm1_call = pl.pallas_call(
    _mm1_body,
    grid=(_NS,),
    in_specs=[
        pl.BlockSpec((_RPT, _DIN), lambda i: (i, 0)),
        pl.BlockSpec((_DIN, _DH), lambda i: (0, 0)),
        pl.BlockSpec((_RPT, _DEGW), lambda i: (i, 0)),
        pl.BlockSpec((_RPT, _DEGW), lambda i: (i, 0)),
    ],
    out_specs=tuple(pl.BlockSpec((_RPT, _QW), lambda i: (i, 0))
                    for _ in range(4)),
    out_shape=tuple(jax.ShapeDtypeStruct((_NP, _QW), jnp.float32)
                    for _ in range(4)),
)


def _mm2_body(s0, s1, s2, s3, d0_ref, d1_ref, b_ref, w_ref, *out_refs):
    i = pl.program_id(0)
    dv = _dinv(i, d0_ref[...], d1_ref[...], _RPT)
    s = jnp.concatenate([s0[...], s1[...], s2[...], s3[...]], axis=1)
    z = jnp.maximum(s * dv + b_ref[...], 0.0)
    h2 = jnp.dot(z, w_ref[...], preferred_element_type=jnp.float32)
    ht2 = h2 * dv
    for q, o_ref in enumerate(out_refs):
        o_ref[...] = ht2[:, q * _QW:(q + 1) * _QW]


_mm2_call = pl.pallas_call(
    _mm2_body,
    grid=(_NS,),
    in_specs=[pl.BlockSpec((_RPT, _QW), lambda i: (i, 0)) for _ in range(4)]
    + [
        pl.BlockSpec((_RPT, _DEGW), lambda i: (i, 0)),
        pl.BlockSpec((_RPT, _DEGW), lambda i: (i, 0)),
        pl.BlockSpec((1, _DH), lambda i: (0, 0)),
        pl.BlockSpec((_DH, _DH), lambda i: (0, 0)),
    ],
    out_specs=tuple(pl.BlockSpec((_RPT, _QW), lambda i: (i, 0))
                    for _ in range(4)),
    out_shape=tuple(jax.ShapeDtypeStruct((_NP, _QW), jnp.float32)
                    for _ in range(4)),
)

_FBLK = 1000


def _fin_body(s0, s1, s2, s3, d0_ref, d1_ref, b_ref, out_ref):
    i = pl.program_id(0)
    dv = _dinv(i, d0_ref[...], d1_ref[...], _FBLK)
    s = jnp.concatenate([s0[...], s1[...], s2[...], s3[...]], axis=1)
    out_ref[...] = jnp.maximum(s * dv + b_ref[...], 0.0)


_fin_call = pl.pallas_call(
    _fin_body,
    grid=(_N // _FBLK,),
    in_specs=[pl.BlockSpec((_FBLK, _QW), lambda i: (i, 0)) for _ in range(4)]
    + [
        pl.BlockSpec((_FBLK, _DEGW), lambda i: (i, 0)),
        pl.BlockSpec((_FBLK, _DEGW), lambda i: (i, 0)),
        pl.BlockSpec((1, _DH), lambda i: (0, 0)),
    ],
    out_specs=pl.BlockSpec((_FBLK, _DH), lambda i: (i, 0)),
    out_shape=jax.ShapeDtypeStruct((_N, _DH), jnp.float32),
)


def kernel(x, edge_index, W1, b1, W2, b2):
    f32 = jnp.float32
    npad = _EP - _E
    padn = _N + (jnp.arange(npad, dtype=jnp.int32) % 224)
    src_p = jnp.concatenate([edge_index[0], padn])
    dst_p = jnp.concatenate([edge_index[1], padn])
    src_m = src_p.reshape(_NS, _MSG_CHUNKS, _CH)
    dst_m = dst_p.reshape(_NS, _MSG_CHUNKS, _CH)
    dst_d = dst_p.reshape(_NC * _NS, _DEG_CHUNKS, _CH)
    x_p = jnp.zeros((_NP, _DIN), f32).at[:_N].set(x)
    ones = jnp.ones((_CH, _DEGW), f32)
    zeros = jnp.zeros((_NP, _DEGW), f32)

    deg0, deg1 = _deg_call()(dst_d, ones, zeros)
    ht1 = _mm1_call(x_p, W1, deg0, deg1)
    s1 = _msg_layer(src_m, dst_m, ht1)
    ht2 = _mm2_call(*s1, deg0, deg1, b1.reshape(1, _DH), W2)
    s2 = _msg_layer(src_m, dst_m, ht2)
    return _fin_call(*s2, deg0, deg1, b2.reshape(1, _DH))


# speedup vs baseline: 1.0395x; 1.0062x over previous
import functools

import jax
import jax.numpy as jnp
from jax import lax
from jax.experimental import pallas as pl
from jax.experimental.pallas import tpu as pltpu
from jax.experimental.pallas import tpu_sc as plsc

_N = 10000
_E = 320000
_DIN = 128
_DH = 256
_QW = 64
_NC = 2
_NS = 16
_CH = 128
_NP = 10240
_EP = 327680
_RPT = _NP // _NS
_MSG_CHUNKS = _EP // (_NS * _CH)
_DEG_CHUNKS = _EP // (_NC * _NS * _CH)
_DEGW = 16


@functools.cache
def _sc_mesh():
    return plsc.VectorSubcoreMesh(core_axis_name="c", subcore_axis_name="s",
                                  num_cores=_NC, num_subcores=_NS)


def _deg_body(dst_hbm, ones_hbm, zeros_hbm, out0_hbm, out1_hbm,
              dstb_v, ones_v, acc_sh, sem):
    cid = lax.axis_index("c")
    sid = lax.axis_index("s")
    wid = cid * _NS + sid
    pltpu.sync_copy(dst_hbm.at[wid], dstb_v)
    pltpu.sync_copy(ones_hbm, ones_v)
    stripe = pl.ds(sid * _RPT, _RPT)
    pltpu.sync_copy(zeros_hbm.at[stripe], acc_sh.at[stripe])
    plsc.subcore_barrier()

    grp = 8

    def group(g, carry):
        ds = []
        for b in range(grp):
            c = g * grp + b
            ds.append(pltpu.async_copy(
                ones_v, acc_sh.at[dstb_v.at[c]], sem, add=True))
        for d in ds:
            d.wait()
        return carry

    lax.fori_loop(0, _DEG_CHUNKS // grp, group, 0)
    plsc.subcore_barrier()

    @pl.when(cid == 0)
    def _():
        pltpu.sync_copy(acc_sh.at[stripe], out0_hbm.at[stripe])

    @pl.when(cid == 1)
    def _():
        pltpu.sync_copy(acc_sh.at[stripe], out1_hbm.at[stripe])


@functools.cache
def _deg_call():
    return pl.kernel(
        _deg_body,
        out_type=(jax.ShapeDtypeStruct((_NP, _DEGW), jnp.float32),
                  jax.ShapeDtypeStruct((_NP, _DEGW), jnp.float32)),
        mesh=_sc_mesh(),
        compiler_params=pltpu.CompilerParams(use_tc_tiling_on_sc=False),
        scratch_types=[
            pltpu.VMEM((_DEG_CHUNKS, _CH), jnp.int32),
            pltpu.VMEM((_CH, _DEGW), jnp.float32),
            pltpu.VMEM_SHARED((_NP, _DEGW), jnp.float32),
            pltpu.SemaphoreType.DMA,
        ],
    )


_NBUF = 4


def _msg_body(src_hbm, dst_hbm, hA_hbm, hB_hbm, outA_hbm, outB_hbm,
              srcb_v, dstb_v, stage_v, acc_sh, gsem, ssem):
    cid = lax.axis_index("c")
    sid = lax.axis_index("s")
    stripe = pl.ds(sid * _RPT, _RPT)
    pltpu.sync_copy(src_hbm.at[sid], srcb_v)
    pltpu.sync_copy(dst_hbm.at[sid], dstb_v)

    def pipeline(h_hbm):
        for b in range(_NBUF):
            pltpu.async_copy(h_hbm.at[srcb_v.at[b]], stage_v.at[b],
                             gsem.at[b])

        def group(k, carry):
            for b in range(_NBUF):
                c = k * _NBUF + b
                pltpu.make_async_copy(h_hbm.at[srcb_v.at[c]], stage_v.at[b],
                                      gsem.at[b]).wait()
                pltpu.async_copy(stage_v.at[b], acc_sh.at[dstb_v.at[c]],
                                 ssem.at[b], add=True).wait()

                @pl.when(c + _NBUF < _MSG_CHUNKS)
                def _():
                    pltpu.async_copy(h_hbm.at[srcb_v.at[c + _NBUF]],
                                     stage_v.at[b], gsem.at[b])
            return carry

        lax.fori_loop(0, _MSG_CHUNKS // _NBUF, group, 0)

    @pl.when(cid == 0)
    def _():
        pltpu.sync_copy(hA_hbm.at[stripe], acc_sh.at[stripe])

    @pl.when(cid == 1)
    def _():
        pltpu.sync_copy(hB_hbm.at[stripe], acc_sh.at[stripe])

    plsc.subcore_barrier()

    @pl.when(cid == 0)
    def _():
        pipeline(hA_hbm)

    @pl.when(cid == 1)
    def _():
        pipeline(hB_hbm)

    plsc.subcore_barrier()

    @pl.when(cid == 0)
    def _():
        pltpu.sync_copy(acc_sh.at[stripe], outA_hbm.at[stripe])

    @pl.when(cid == 1)
    def _():
        pltpu.sync_copy(acc_sh.at[stripe], outB_hbm.at[stripe])


@functools.cache
def _msg_call():
    return pl.kernel(
        _msg_body,
        out_type=(jax.ShapeDtypeStruct((_NP, _QW), jnp.float32),
                  jax.ShapeDtypeStruct((_NP, _QW), jnp.float32)),
        mesh=_sc_mesh(),
        compiler_params=pltpu.CompilerParams(use_tc_tiling_on_sc=False),
        scratch_types=[
            pltpu.VMEM((_MSG_CHUNKS, _CH), jnp.int32),
            pltpu.VMEM((_MSG_CHUNKS, _CH), jnp.int32),
            pltpu.VMEM((_NBUF, _CH, _QW), jnp.float32),
            pltpu.VMEM_SHARED((_NP, _QW), jnp.float32),
            pltpu.SemaphoreType.DMA((_NBUF,)),
            pltpu.SemaphoreType.DMA((_NBUF,)),
        ],
    )


def _msg_layer(src_m, dst_m, hq):
    s0, s1 = _msg_call()(src_m, dst_m, hq[0], hq[1])
    s2, s3 = _msg_call()(src_m, dst_m, hq[2], hq[3])
    return (s0, s1, s2, s3)


def _dinv(i, d0, d1, blk):
    deg = d0[:, :1] + d1[:, :1] + 1.0
    rows = i * blk + lax.broadcasted_iota(jnp.int32, (blk, 1), 0)
    return jnp.where(rows < _N, lax.rsqrt(deg), 0.0)


def _mm1_body(x_ref, w_ref, d0_ref, d1_ref, *out_refs):
    i = pl.program_id(0)
    h = jnp.dot(x_ref[...], w_ref[...], preferred_element_type=jnp.float32)
    ht = h * _dinv(i, d0_ref[...], d1_ref[...], _RPT)
    for q, o_ref in enumerate(out_refs):
        o_ref[...] = ht[:, q * _QW:(q + 1) * _QW]


_mm1_call = pl.pallas_call(
    _mm1_body,
    grid=(_NS,),
    in_specs=[
        pl.BlockSpec((_RPT, _DIN), lambda i: (i, 0)),
        pl.BlockSpec((_DIN, _DH), lambda i: (0, 0)),
        pl.BlockSpec((_RPT, _DEGW), lambda i: (i, 0)),
        pl.BlockSpec((_RPT, _DEGW), lambda i: (i, 0)),
    ],
    out_specs=tuple(pl.BlockSpec((_RPT, _QW), lambda i: (i, 0))
                    for _ in range(4)),
    out_shape=tuple(jax.ShapeDtypeStruct((_NP, _QW), jnp.float32)
                    for _ in range(4)),
)


def _mm2_body(s0, s1, s2, s3, d0_ref, d1_ref, b_ref, w_ref, *out_refs):
    i = pl.program_id(0)
    dv = _dinv(i, d0_ref[...], d1_ref[...], _RPT)
    s = jnp.concatenate([s0[...], s1[...], s2[...], s3[...]], axis=1)
    z = jnp.maximum(s * dv + b_ref[...], 0.0)
    h2 = jnp.dot(z, w_ref[...], preferred_element_type=jnp.float32)
    ht2 = h2 * dv
    for q, o_ref in enumerate(out_refs):
        o_ref[...] = ht2[:, q * _QW:(q + 1) * _QW]


_mm2_call = pl.pallas_call(
    _mm2_body,
    grid=(_NS,),
    in_specs=[pl.BlockSpec((_RPT, _QW), lambda i: (i, 0)) for _ in range(4)]
    + [
        pl.BlockSpec((_RPT, _DEGW), lambda i: (i, 0)),
        pl.BlockSpec((_RPT, _DEGW), lambda i: (i, 0)),
        pl.BlockSpec((1, _DH), lambda i: (0, 0)),
        pl.BlockSpec((_DH, _DH), lambda i: (0, 0)),
    ],
    out_specs=tuple(pl.BlockSpec((_RPT, _QW), lambda i: (i, 0))
                    for _ in range(4)),
    out_shape=tuple(jax.ShapeDtypeStruct((_NP, _QW), jnp.float32)
                    for _ in range(4)),
)

_FBLK = 1000


def _fin_body(s0, s1, s2, s3, d0_ref, d1_ref, b_ref, out_ref):
    i = pl.program_id(0)
    dv = _dinv(i, d0_ref[...], d1_ref[...], _FBLK)
    s = jnp.concatenate([s0[...], s1[...], s2[...], s3[...]], axis=1)
    out_ref[...] = jnp.maximum(s * dv + b_ref[...], 0.0)


_fin_call = pl.pallas_call(
    _fin_body,
    grid=(_N // _FBLK,),
    in_specs=[pl.BlockSpec((_FBLK, _QW), lambda i: (i, 0)) for _ in range(4)]
    + [
        pl.BlockSpec((_FBLK, _DEGW), lambda i: (i, 0)),
        pl.BlockSpec((_FBLK, _DEGW), lambda i: (i, 0)),
        pl.BlockSpec((1, _DH), lambda i: (0, 0)),
    ],
    out_specs=pl.BlockSpec((_FBLK, _DH), lambda i: (i, 0)),
    out_shape=jax.ShapeDtypeStruct((_N, _DH), jnp.float32),
)


def kernel(x, edge_index, W1, b1, W2, b2):
    f32 = jnp.float32
    npad = _EP - _E
    padn = _N + (jnp.arange(npad, dtype=jnp.int32) % 224)
    src_p = jnp.concatenate([edge_index[0], padn])
    dst_p = jnp.concatenate([edge_index[1], padn])
    src_m = src_p.reshape(_NS, _MSG_CHUNKS, _CH)
    dst_m = dst_p.reshape(_NS, _MSG_CHUNKS, _CH)
    dst_d = dst_p.reshape(_NC * _NS, _DEG_CHUNKS, _CH)
    x_p = jnp.zeros((_NP, _DIN), f32).at[:_N].set(x)
    ones = jnp.ones((_CH, _DEGW), f32)
    zeros = jnp.zeros((_NP, _DEGW), f32)

    deg0, deg1 = _deg_call()(dst_d, ones, zeros)
    ht1 = _mm1_call(x_p, W1, deg0, deg1)
    s1 = _msg_layer(src_m, dst_m, ht1)
    ht2 = _mm2_call(*s1, deg0, deg1, b1.reshape(1, _DH), W2)
    s2 = _msg_layer(src_m, dst_m, ht2)
    return _fin_call(*s2, deg0, deg1, b2.reshape(1, _DH))
